# Initial kernel scaffold; baseline (speedup 1.0000x reference)
import jax
import jax.numpy as jnp
from jax.experimental import pallas as pl


def kernel(z, W):
    raise NotImplementedError("write your pallas kernel here")


# speedup vs baseline: 1.2063x; 1.2063x over previous
import jax
import jax.numpy as jnp
from jax.experimental import pallas as pl


def _vq_kernel(z_ref, w_ref, q_ref, loss_ref, cl_ref, el_ref):
    zb = z_ref[0]
    W = w_ref[...]
    D, N = zb.shape
    K = W.shape[1]

    zt = zb.T
    wt = W.T

    S = jax.lax.dot_general(zt, W, (((1,), (0,)), ((), ())),
                            preferred_element_type=jnp.float32)
    zsq = jnp.sum(zt * zt, axis=1)
    wsq = jnp.sum(wt * wt, axis=1)
    d = (zsq[:, None] + wsq[None, :]) - 2.0 * S

    m = jnp.min(d, axis=1)
    iota_k = jax.lax.broadcasted_iota(jnp.int32, d.shape, 1)
    closest = jnp.min(jnp.where(d == m[:, None], iota_k, jnp.int32(K)),
                      axis=1)

    kio = jax.lax.broadcasted_iota(jnp.int32, (K, N), 0)
    onehot = (kio == closest[None, :]).astype(jnp.float32)
    qb = jax.lax.dot_general(W, onehot, (((1,), (0,)), ((), ())),
                             precision=jax.lax.Precision.HIGHEST,
                             preferred_element_type=jnp.float32)

    q_ref[0] = zb + (qb - zb)
    diff = qb - zb
    c = jnp.sum(diff * diff, axis=0) * jnp.float32(1.0 / D)
    cl_ref[0, 0] = c
    el_ref[0, 0] = c
    loss_ref[0, 0] = c * jnp.float32(0.25) + c


def kernel(z, W):
    B, D, H, Wd = z.shape
    N = H * Wd
    K = W.shape[1]
    z3 = z.reshape(B, D, N)
    f32 = jnp.float32
    q, loss, cl, el = pl.pallas_call(
        _vq_kernel,
        grid=(B,),
        in_specs=[
            pl.BlockSpec((1, D, N), lambda b: (b, 0, 0)),
            pl.BlockSpec((D, K), lambda b: (0, 0)),
        ],
        out_specs=[
            pl.BlockSpec((1, D, N), lambda b: (b, 0, 0)),
            pl.BlockSpec((1, 1, N), lambda b: (b, 0, 0)),
            pl.BlockSpec((1, 1, N), lambda b: (b, 0, 0)),
            pl.BlockSpec((1, 1, N), lambda b: (b, 0, 0)),
        ],
        out_shape=[
            jax.ShapeDtypeStruct((B, D, N), f32),
            jax.ShapeDtypeStruct((B, 1, N), f32),
            jax.ShapeDtypeStruct((B, 1, N), f32),
            jax.ShapeDtypeStruct((B, 1, N), f32),
        ],
    )(z3, W)
    shp = (B, H, Wd)
    return (q.reshape(z.shape), loss.reshape(shp), cl.reshape(shp),
            el.reshape(shp))


# speedup vs baseline: 1.8145x; 1.5042x over previous
import jax
import jax.numpy as jnp
from jax.experimental import pallas as pl
from jax.experimental.pallas import tpu as pltpu


def _vq_kernel(z_ref, w_ref, q_ref, loss_ref, cl_ref, el_ref):
    zb = z_ref[0]
    W = w_ref[...]
    D, N = zb.shape
    K = W.shape[1]

    zt = zb.T
    wt = W.T

    S2 = jax.lax.dot_general(zt * jnp.float32(-2.0), W,
                             (((1,), (0,)), ((), ())),
                             preferred_element_type=jnp.float32)
    zsq = jnp.sum(zt * zt, axis=1)
    wsq = jnp.sum(wt * wt, axis=1)
    d = (zsq[:, None] + wsq[None, :]) + S2

    m = jnp.min(d, axis=1)
    iota_k = jax.lax.broadcasted_iota(jnp.int32, d.shape, 1)
    closest = jnp.min(jnp.where(d == m[:, None], iota_k, jnp.int32(K)),
                      axis=1)

    kio = jax.lax.broadcasted_iota(jnp.int32, (K, N), 0)
    onehot = (kio == closest[None, :]).astype(jnp.float32)
    qb = jax.lax.dot_general(W, onehot, (((1,), (0,)), ((), ())),
                             preferred_element_type=jnp.float32)

    q_ref[0] = zb + (qb - zb)
    diff = qb - zb
    c = jnp.sum(diff * diff, axis=0) * jnp.float32(1.0 / D)
    cl_ref[0, 0] = c
    el_ref[0, 0] = c
    loss_ref[0, 0] = c * jnp.float32(0.25) + c


def kernel(z, W):
    B, D, H, Wd = z.shape
    N = H * Wd
    K = W.shape[1]
    z3 = z.reshape(B, D, N)
    f32 = jnp.float32
    q, loss, cl, el = pl.pallas_call(
        _vq_kernel,
        grid=(B,),
        in_specs=[
            pl.BlockSpec((1, D, N), lambda b: (b, 0, 0)),
            pl.BlockSpec((D, K), lambda b: (0, 0)),
        ],
        out_specs=[
            pl.BlockSpec((1, D, N), lambda b: (b, 0, 0)),
            pl.BlockSpec((1, 1, N), lambda b: (b, 0, 0)),
            pl.BlockSpec((1, 1, N), lambda b: (b, 0, 0)),
            pl.BlockSpec((1, 1, N), lambda b: (b, 0, 0)),
        ],
        out_shape=[
            jax.ShapeDtypeStruct((B, D, N), f32),
            jax.ShapeDtypeStruct((B, 1, N), f32),
            jax.ShapeDtypeStruct((B, 1, N), f32),
            jax.ShapeDtypeStruct((B, 1, N), f32),
        ],
        compiler_params=pltpu.CompilerParams(
            dimension_semantics=("parallel",)),
    )(z3, W)
    shp = (B, H, Wd)
    return (q.reshape(z.shape), loss.reshape(shp), cl.reshape(shp),
            el.reshape(shp))


# speedup vs baseline: 2.0135x; 1.1097x over previous
import jax
import jax.numpy as jnp
from jax.experimental import pallas as pl
from jax.experimental.pallas import tpu as pltpu


def _vq_kernel(z_ref, w_ref, q_ref, loss_ref, cl_ref, el_ref):
    zb = z_ref[0]
    W = w_ref[...]
    D, N = zb.shape
    K = W.shape[1]

    zt = zb.T
    wt = W.T

    S2 = jax.lax.dot_general(zt * jnp.float32(-2.0), W,
                             (((1,), (0,)), ((), ())),
                             preferred_element_type=jnp.float32)
    zsq = jnp.sum(zt * zt, axis=1)
    wsq = jnp.sum(wt * wt, axis=1)
    d = (zsq[:, None] + wsq[None, :]) + S2

    m = jnp.min(d, axis=1)
    iota_row = jax.lax.broadcasted_iota(jnp.int32, (1, K), 1).astype(
        jnp.float32)
    closest = jnp.min(jnp.where(d == m[:, None], iota_row,
                                jnp.float32(K)), axis=1)

    kcol = jax.lax.broadcasted_iota(jnp.int32, (K, 1), 0).astype(
        jnp.float32)
    onehot = (kcol == closest[None, :]).astype(jnp.float32)
    qb = jax.lax.dot_general(W, onehot, (((1,), (0,)), ((), ())),
                             preferred_element_type=jnp.float32)

    q_ref[0] = zb + (qb - zb)
    diff = qb - zb
    c = jnp.sum(diff * diff, axis=0) * jnp.float32(1.0 / D)
    cl_ref[0, 0] = c
    el_ref[0, 0] = c
    loss_ref[0, 0] = c * jnp.float32(0.25) + c


def kernel(z, W):
    B, D, H, Wd = z.shape
    N = H * Wd
    K = W.shape[1]
    z3 = z.reshape(B, D, N)
    f32 = jnp.float32
    q, loss, cl, el = pl.pallas_call(
        _vq_kernel,
        grid=(B,),
        in_specs=[
            pl.BlockSpec((1, D, N), lambda b: (b, 0, 0)),
            pl.BlockSpec((D, K), lambda b: (0, 0)),
        ],
        out_specs=[
            pl.BlockSpec((1, D, N), lambda b: (b, 0, 0)),
            pl.BlockSpec((1, 1, N), lambda b: (b, 0, 0)),
            pl.BlockSpec((1, 1, N), lambda b: (b, 0, 0)),
            pl.BlockSpec((1, 1, N), lambda b: (b, 0, 0)),
        ],
        out_shape=[
            jax.ShapeDtypeStruct((B, D, N), f32),
            jax.ShapeDtypeStruct((B, 1, N), f32),
            jax.ShapeDtypeStruct((B, 1, N), f32),
            jax.ShapeDtypeStruct((B, 1, N), f32),
        ],
        compiler_params=pltpu.CompilerParams(
            dimension_semantics=("parallel",)),
    )(z3, W)
    shp = (B, H, Wd)
    return (q.reshape(z.shape), loss.reshape(shp), cl.reshape(shp),
            el.reshape(shp))
